# baseline (device time: 86848 ns/iter reference)
import jax
import jax.numpy as jnp
from jax import lax
from jax.experimental import pallas as pl
from jax.experimental.pallas import tpu as pltpu

N_DEV = 8
B, SQ, D_MODEL = 2, 128, 512
HQ_LOCAL, DH = 4, 64
SKV_LIVE = 128
BLK = 64
SCALE = 0.125
BFLY_XOR = (1, 3, 4)


def kernel(x, Wq, K_ext, V_ext, Wo):
    def body(x_ref, wq_ref, k_ref, v_ref, wo_ref, out_ref,
             kbuf, vbuf, acc_ref, rbuf,
             kv_send_sems, kv_recv_sems, bf_send_sems, bf_recv_sems,
             ack_sem):
        me = lax.axis_index("i")

        bar = pltpu.get_barrier_semaphore()
        for d in range(N_DEV):
            @pl.when(me != d)
            def _():
                pl.semaphore_signal(
                    bar, inc=1, device_id=(d,),
                    device_id_type=pl.DeviceIdType.MESH,
                )
        pl.semaphore_wait(bar, N_DEV - 1)

        def kv_rdma(p, t, src_ref, dst_ref):
            return pltpu.make_async_remote_copy(
                src_ref=src_ref.at[:, :, pl.ds(HQ_LOCAL * p, HQ_LOCAL), :],
                dst_ref=dst_ref,
                send_sem=kv_send_sems.at[max(p - 1, 0), t],
                recv_sem=kv_recv_sems.at[t],
                device_id=(p,),
                device_id_type=pl.DeviceIdType.MESH,
            )

        @pl.when(me == 0)
        def _():
            for p in range(1, N_DEV):
                kv_rdma(p, 0, k_ref, kbuf).start()
                kv_rdma(p, 1, v_ref, vbuf).start()
            kbuf[...] = k_ref[:, :, 0:HQ_LOCAL, :]
            vbuf[...] = v_ref[:, :, 0:HQ_LOCAL, :]

        x2d = x_ref[...].reshape(B * SQ, D_MODEL)
        q2d = jnp.dot(x2d, wq_ref[...], preferred_element_type=jnp.float32)

        @pl.when(me != 0)
        def _():
            kv_rdma(0, 0, k_ref, kbuf).wait_recv()
            kv_rdma(0, 1, v_ref, vbuf).wait_recv()
            pl.semaphore_signal(
                ack_sem, inc=1, device_id=(0,),
                device_id_type=pl.DeviceIdType.MESH,
            )

        qi = lax.broadcasted_iota(jnp.int32, (SQ, SKV_LIVE), 0) // BLK
        kj = lax.broadcasted_iota(jnp.int32, (SQ, SKV_LIVE), 1) // BLK
        mask = kj <= qi

        wo = wo_ref[...]
        for b in range(B):
            acc_b = jnp.zeros((SQ, D_MODEL), jnp.float32)
            for h in range(HQ_LOCAL):
                q_bh = q2d[b * SQ:(b + 1) * SQ, h * DH:(h + 1) * DH]
                k_bh = kbuf[b, :, h, :]
                v_bh = vbuf[b, :, h, :]
                s = lax.dot_general(
                    q_bh, k_bh, (((1,), (1,)), ((), ())),
                    preferred_element_type=jnp.float32,
                ) * SCALE
                s = jnp.where(mask, s, -1e9)
                m = jnp.max(s, axis=-1, keepdims=True)
                w = jnp.exp(s - m)
                w = w / jnp.sum(w, axis=-1, keepdims=True)
                ctx = jnp.dot(w, v_bh, preferred_element_type=jnp.float32)
                acc_b = acc_b + jnp.dot(
                    ctx, wo[h * DH:(h + 1) * DH, :],
                    preferred_element_type=jnp.float32,
                )
            acc_ref[b * SQ:(b + 1) * SQ, :] = acc_b

        @pl.when(me == 0)
        def _():
            for p in range(1, N_DEV):
                kv_rdma(p, 0, k_ref, kbuf).wait_send()
                kv_rdma(p, 1, v_ref, vbuf).wait_send()

        for r, xr in enumerate(BFLY_XOR):
            partner = me ^ xr
            ex = pltpu.make_async_remote_copy(
                src_ref=acc_ref,
                dst_ref=rbuf.at[r],
                send_sem=bf_send_sems.at[r],
                recv_sem=bf_recv_sems.at[r],
                device_id=(partner,),
                device_id_type=pl.DeviceIdType.MESH,
            )
            ex.start()
            ex.wait()
            acc_ref[...] = acc_ref[...] + rbuf[r]

        @pl.when(me == 0)
        def _():
            pl.semaphore_wait(ack_sem, N_DEV - 1)

        out_ref[...] = acc_ref[...].reshape(B, SQ, D_MODEL)

    return pl.pallas_call(
        body,
        out_shape=jax.ShapeDtypeStruct((B, SQ, D_MODEL), jnp.float32),
        in_specs=[pl.BlockSpec(memory_space=pltpu.VMEM)] * 5,
        out_specs=pl.BlockSpec(memory_space=pltpu.VMEM),
        scratch_shapes=[
            pltpu.VMEM((B, SKV_LIVE, HQ_LOCAL, DH), jnp.float32),
            pltpu.VMEM((B, SKV_LIVE, HQ_LOCAL, DH), jnp.float32),
            pltpu.VMEM((B * SQ, D_MODEL), jnp.float32),
            pltpu.VMEM((3, B * SQ, D_MODEL), jnp.float32),
            pltpu.SemaphoreType.DMA((N_DEV - 1, 2)),
            pltpu.SemaphoreType.DMA((2,)),
            pltpu.SemaphoreType.DMA((3,)),
            pltpu.SemaphoreType.DMA((3,)),
            pltpu.SemaphoreType.REGULAR,
        ],
        compiler_params=pltpu.CompilerParams(collective_id=0),
    )(x, Wq, K_ext, V_ext, Wo)


# device time: 56345 ns/iter; 1.5414x vs baseline; 1.5414x over previous
import jax
import jax.numpy as jnp
from jax import lax
from jax.experimental import pallas as pl
from jax.experimental.pallas import tpu as pltpu

N_DEV = 8
B, SQ, D_MODEL = 2, 128, 512
HQ_LOCAL, DH = 4, 64
SKV_LIVE = 128
BLK = 64
SCALE = 0.125
BFLY_XOR = (1, 3, 4)


def kernel(x, Wq, K_ext, V_ext, Wo):
    def body(x_ref, wq_ref, k_ref, v_ref, wo_ref, out_ref,
             kv_stage, kvbuf, acc_ref, sbuf, rbuf,
             kv_send_sems, kv_recv_sem, bf_send_sems, bf_recv_sems,
             ack_sem):
        me = lax.axis_index("i")

        bar = pltpu.get_barrier_semaphore()
        for d in range(N_DEV):
            @pl.when(me != d)
            def _():
                pl.semaphore_signal(
                    bar, inc=1, device_id=(d,),
                    device_id_type=pl.DeviceIdType.MESH,
                )
        pl.semaphore_wait(bar, N_DEV - 1)

        def kv_rdma(p):
            return pltpu.make_async_remote_copy(
                src_ref=kv_stage.at[max(p - 1, 0)],
                dst_ref=kvbuf,
                send_sem=kv_send_sems.at[max(p - 1, 0)],
                recv_sem=kv_recv_sem,
                device_id=(p,),
                device_id_type=pl.DeviceIdType.MESH,
            )

        @pl.when(me == 0)
        def _():
            for p in range(1, N_DEV):
                s = pl.ds(HQ_LOCAL * p, HQ_LOCAL)
                kv_stage[p - 1, 0] = k_ref[:, :, s, :].astype(jnp.bfloat16)
                kv_stage[p - 1, 1] = v_ref[:, :, s, :].astype(jnp.bfloat16)
                kv_rdma(p).start()
            kvbuf[0] = k_ref[:, :, 0:HQ_LOCAL, :].astype(jnp.bfloat16)
            kvbuf[1] = v_ref[:, :, 0:HQ_LOCAL, :].astype(jnp.bfloat16)

        x2d = x_ref[...].reshape(B * SQ, D_MODEL)
        q2d = jnp.dot(x2d, wq_ref[...], preferred_element_type=jnp.float32)
        q2d = q2d.astype(jnp.bfloat16)

        @pl.when(me != 0)
        def _():
            kv_rdma(0).wait_recv()
            pl.semaphore_signal(
                ack_sem, inc=1, device_id=(0,),
                device_id_type=pl.DeviceIdType.MESH,
            )

        qi = lax.broadcasted_iota(jnp.int32, (SQ, SKV_LIVE), 0) // BLK
        kj = lax.broadcasted_iota(jnp.int32, (SQ, SKV_LIVE), 1) // BLK
        mask = kj <= qi

        wo = wo_ref[...]
        for b in range(B):
            acc_b = jnp.zeros((SQ, D_MODEL), jnp.float32)
            for h in range(HQ_LOCAL):
                q_bh = q2d[b * SQ:(b + 1) * SQ, h * DH:(h + 1) * DH]
                k_bh = kvbuf[0, b, :, h, :]
                v_bh = kvbuf[1, b, :, h, :]
                s = lax.dot_general(
                    q_bh, k_bh, (((1,), (1,)), ((), ())),
                    preferred_element_type=jnp.float32,
                ) * SCALE
                s = jnp.where(mask, s, -1e9)
                m = jnp.max(s, axis=-1, keepdims=True)
                w = jnp.exp(s - m)
                w = (w / jnp.sum(w, axis=-1, keepdims=True)).astype(jnp.bfloat16)
                ctx = jnp.dot(w, v_bh, preferred_element_type=jnp.float32)
                acc_b = acc_b + jnp.dot(
                    ctx, wo[h * DH:(h + 1) * DH, :],
                    preferred_element_type=jnp.float32,
                )
            acc_ref[b * SQ:(b + 1) * SQ, :] = acc_b

        @pl.when(me == 0)
        def _():
            for p in range(1, N_DEV):
                kv_rdma(p).wait_send()

        for r, xr in enumerate(BFLY_XOR):
            partner = me ^ xr
            sbuf[...] = acc_ref[...].astype(jnp.bfloat16)
            ex = pltpu.make_async_remote_copy(
                src_ref=sbuf,
                dst_ref=rbuf.at[r],
                send_sem=bf_send_sems.at[r],
                recv_sem=bf_recv_sems.at[r],
                device_id=(partner,),
                device_id_type=pl.DeviceIdType.MESH,
            )
            ex.start()
            ex.wait()
            acc_ref[...] = acc_ref[...] + rbuf[r].astype(jnp.float32)

        @pl.when(me == 0)
        def _():
            pl.semaphore_wait(ack_sem, N_DEV - 1)

        out_ref[...] = acc_ref[...].reshape(B, SQ, D_MODEL)

    kv_slice = (B, SKV_LIVE, HQ_LOCAL, DH)
    return pl.pallas_call(
        body,
        out_shape=jax.ShapeDtypeStruct((B, SQ, D_MODEL), jnp.float32),
        in_specs=[pl.BlockSpec(memory_space=pltpu.VMEM)] * 5,
        out_specs=pl.BlockSpec(memory_space=pltpu.VMEM),
        scratch_shapes=[
            pltpu.VMEM((N_DEV - 1, 2) + kv_slice, jnp.bfloat16),
            pltpu.VMEM((2,) + kv_slice, jnp.bfloat16),
            pltpu.VMEM((B * SQ, D_MODEL), jnp.float32),
            pltpu.VMEM((B * SQ, D_MODEL), jnp.bfloat16),
            pltpu.VMEM((3, B * SQ, D_MODEL), jnp.bfloat16),
            pltpu.SemaphoreType.DMA((N_DEV - 1,)),
            pltpu.SemaphoreType.DMA,
            pltpu.SemaphoreType.DMA((3,)),
            pltpu.SemaphoreType.DMA((3,)),
            pltpu.SemaphoreType.REGULAR,
        ],
        compiler_params=pltpu.CompilerParams(collective_id=0),
    )(x, Wq, K_ext, V_ext, Wo)
